# Initial kernel scaffold; baseline (speedup 1.0000x reference)
import jax
import jax.numpy as jnp
from jax.experimental import pallas as pl


def kernel(x, edge_index, edge_attr, edge_weights_index, conv_z_W, conv_z_b, lin_z_W, lin_z_b, conv_r_W, conv_r_b, lin_r_W, lin_r_b, conv_h_W, conv_h_b, lin_h_W, lin_h_b, attention, dec1_W, dec1_b, dec2_W, dec2_b):
    raise NotImplementedError("write your pallas kernel here")


# speedup vs baseline: 9.7062x; 9.7062x over previous
import functools

import jax
import jax.numpy as jnp
from jax import lax
from jax.experimental import pallas as pl
from jax.experimental.pallas import tpu as pltpu
from jax.experimental.pallas import tpu_sc as plsc

L = 16
NC = 2
NS = 16
NW = NC * NS
D = 128
SL = D // L


def _rsqrt_pos(d):
    bits = lax.bitcast_convert_type(d, jnp.int32)
    y = lax.bitcast_convert_type(jnp.int32(0x5F3759DF) - (bits >> 1),
                                 jnp.float32)
    for _ in range(3):
        y = y * (1.5 - 0.5 * d * y * y)
    return jnp.where(d > 0.0, y, 0.0)


def _gcn_agg_body(src_h, dst_h, ew_h, x_h, agg_h,
                  src_v, dst_v, ew_v, rows_a, rows_b, normb, dinv_v, stripe_v,
                  deg_sh, agg_sh, sem_a, sem_b, sem_s):
    c = lax.axis_index("c")
    s = lax.axis_index("s")
    w = c * NS + s
    rpt = src_h.shape[0] // NW
    brk = src_v.shape[0]
    npad = deg_sh.shape[0]
    stride = npad // NS

    zv = jnp.zeros((L,), jnp.float32)

    def zrows(i, _):
        for u in range(SL):
            rows_a[i, pl.ds(u * L, L)] = zv
        return 0
    lax.fori_loop(0, D, zrows, 0)

    def zstripe(i, _):
        stripe_v[pl.ds(i * L, L)] = zv
        return 0
    lax.fori_loop(0, stride // L, zstripe, 0)

    pltpu.sync_copy(stripe_v, deg_sh.at[pl.ds(s * stride, stride)])
    for k in range(stride // D):
        pltpu.sync_copy(rows_a, agg_sh.at[pl.ds(s * stride + k * D, D)])
    plsc.subcore_barrier()

    def deg_blk(b, _):
        base = s * (2 * rpt) + b * brk
        pltpu.sync_copy(dst_h.at[pl.ds(base, brk)], dst_v)
        pltpu.sync_copy(ew_h.at[pl.ds(base, brk)], ew_v)
        descs = [
            pltpu.async_copy(ew_v.at[j], deg_sh.at[dst_v.at[j]],
                             sem_s, add=True)
            for j in range(brk)
        ]
        for dsc in descs:
            dsc.wait()
        return 0
    lax.fori_loop(0, (2 * rpt) // brk, deg_blk, 0)
    plsc.subcore_barrier()

    pltpu.sync_copy(deg_sh.at[pl.ds(s * stride, stride)], stripe_v)

    def dinv_loop(i, _):
        stripe_v[pl.ds(i * L, L)] = _rsqrt_pos(stripe_v[pl.ds(i * L, L)])
        return 0
    lax.fori_loop(0, stride // L, dinv_loop, 0)
    pltpu.sync_copy(stripe_v, deg_sh.at[pl.ds(s * stride, stride)])
    plsc.subcore_barrier()
    pltpu.sync_copy(deg_sh, dinv_v)

    bufs = (rows_a, rows_b)
    sems = (sem_a, sem_b)

    def agg_blk(b, _):
        base = w * rpt + b * brk
        pltpu.sync_copy(src_h.at[pl.ds(base, brk)], src_v)
        pltpu.sync_copy(dst_h.at[pl.ds(base, brk)], dst_v)
        pltpu.sync_copy(ew_h.at[pl.ds(base, brk)], ew_v)
        pltpu.async_copy(x_h.at[src_v.at[0]], rows_a, sem_a)
        for j in range(brk):
            rows = bufs[j % 2]
            pltpu.make_async_copy(x_h.at[src_v.at[j]], rows, sems[j % 2]).wait()
            if j + 1 < brk:
                pltpu.async_copy(x_h.at[src_v.at[j + 1]], bufs[(j + 1) % 2],
                                 sems[(j + 1) % 2])
            for u in range(SL):
                sv = src_v[j, pl.ds(u * L, L)]
                dv = dst_v[j, pl.ds(u * L, L)]
                evv = ew_v[j, pl.ds(u * L, L)]
                d1 = plsc.load_gather(dinv_v, [sv])
                d2 = plsc.load_gather(dinv_v, [dv])
                normb[pl.ds(u * L, L)] = d1 * evv * d2

            def scale(e, _, rows=rows):
                nr = plsc.load_gather(normb, [jnp.broadcast_to(e, (L,))])
                for u in range(SL):
                    rows[e, pl.ds(u * L, L)] = rows[e, pl.ds(u * L, L)] * nr
                return 0
            lax.fori_loop(0, D, scale, 0)
            pltpu.sync_copy(rows, agg_sh.at[dst_v.at[j]], add=True)
        return 0
    lax.fori_loop(0, rpt // brk, agg_blk, 0)

    plsc.subcore_barrier()
    pltpu.sync_copy(agg_sh.at[pl.ds(s * stride, stride)],
                    agg_h.at[c, pl.ds(s * stride, stride)])


def _dense_body(agg_ref, czW_ref, lzW_ref, lzb_ref, czb_ref,
                chW_ref, lhW_ref, lhb_ref, chb_ref, d1W_ref, d1b_ref,
                p1_ref, p2_ref):
    f32 = jnp.float32
    agg = agg_ref[0] + agg_ref[1]
    mz = jnp.dot(czW_ref[...], lzW_ref[0:D, :], preferred_element_type=f32)
    cz = jnp.dot(czb_ref[...], lzW_ref[0:D, :], preferred_element_type=f32) + lzb_ref[...]
    mh = jnp.dot(chW_ref[...], lhW_ref[0:D, :], preferred_element_type=f32)
    ch = jnp.dot(chb_ref[...], lhW_ref[0:D, :], preferred_element_type=f32) + lhb_ref[...]
    z = jax.nn.sigmoid(jnp.dot(agg, mz, preferred_element_type=f32) + cz)
    ht = jnp.tanh(jnp.dot(agg, mh, preferred_element_type=f32) + ch)
    emb = (1.0 - z) * ht
    p1_ref[...] = jnp.dot(emb, d1W_ref[0:D, :], preferred_element_type=f32) + d1b_ref[...]
    p2_ref[...] = jnp.dot(emb, d1W_ref[D:2 * D, :], preferred_element_type=f32)


def _dec_body(p1_h, p2_h, s_h, t_h, w2_h, b_h, out_h,
              s_v, t_v, r1a, r1b, r2a, r2b, predb, w2_v, b_v,
              sem1a, sem1b, sem2a, sem2b):
    c = lax.axis_index("c")
    s = lax.axis_index("s")
    w = c * NS + s
    rpt = s_h.shape[0] // NW
    base = w * rpt
    pltpu.sync_copy(s_h.at[pl.ds(base, rpt)], s_v)
    pltpu.sync_copy(t_h.at[pl.ds(base, rpt)], t_v)
    pltpu.sync_copy(w2_h, w2_v)
    pltpu.sync_copy(b_h, b_v)
    wsl = [w2_v[pl.ds(u * L, L)] for u in range(SL)]
    bsc = b_v[...][0]
    lane = lax.iota(jnp.int32, L)

    pltpu.async_copy(p1_h.at[s_v.at[0]], r1a, sem1a)
    pltpu.async_copy(p2_h.at[t_v.at[0]], r2a, sem2a)
    pltpu.async_copy(p1_h.at[s_v.at[1]], r1b, sem1b)
    pltpu.async_copy(p2_h.at[t_v.at[1]], r2b, sem2b)

    def process(j, r1, r2, sem1, sem2):
        pltpu.make_async_copy(p1_h.at[s_v.at[j]], r1, sem1).wait()
        pltpu.make_async_copy(p2_h.at[t_v.at[j]], r2, sem2).wait()

        def group(g, _):
            out_v = jnp.zeros((L,), jnp.float32)
            for i in range(L):
                e = g * L + i
                acc = jnp.zeros((L,), jnp.float32)
                for u in range(SL):
                    v = r1[e, pl.ds(u * L, L)] + r2[e, pl.ds(u * L, L)]
                    acc = acc + jnp.maximum(v, 0.0) * wsl[u]
                out_v = jnp.where(lane == i, jnp.sum(acc) + bsc, out_v)
            predb[j, pl.ds(g * L, L)] = out_v
            return 0
        lax.fori_loop(0, D // L, group, 0)

    def loop(jj, _):
        j0 = jj * 2
        process(j0, r1a, r2a, sem1a, sem2a)

        @pl.when(jj < rpt // 2 - 1)
        def _():
            pltpu.async_copy(p1_h.at[s_v.at[j0 + 2]], r1a, sem1a)
            pltpu.async_copy(p2_h.at[t_v.at[j0 + 2]], r2a, sem2a)

        process(j0 + 1, r1b, r2b, sem1b, sem2b)

        @pl.when(jj < rpt // 2 - 1)
        def _():
            pltpu.async_copy(p1_h.at[s_v.at[j0 + 3]], r1b, sem1b)
            pltpu.async_copy(p2_h.at[t_v.at[j0 + 3]], r2b, sem2b)
        return 0
    lax.fori_loop(0, rpt // 2, loop, 0)

    pltpu.sync_copy(predb, out_h.at[pl.ds(w * rpt, rpt)])


def kernel(x, edge_index, edge_attr, edge_weights_index, conv_z_W, conv_z_b,
           lin_z_W, lin_z_b, conv_r_W, conv_r_b, lin_r_W, lin_r_b, conv_h_W,
           conv_h_b, lin_h_W, lin_h_b, attention, dec1_W, dec1_b, dec2_W,
           dec2_b):
    f32 = jnp.float32
    n = x.shape[0]
    e = edge_index.shape[1]

    npad = ((n + NS * L - 1) // (NS * L)) * (NS * L)
    rpt = -(-e // (NW * 128))
    rpt = rpt + (rpt % 2)
    e_pad = NW * rpt * 128
    er = e_pad // 128

    def pad2d(a):
        return jnp.pad(a, (0, e_pad - e)).reshape(er, 128)

    src_p = pad2d(edge_index[0])
    dst_p = pad2d(edge_index[1])
    ew_p = pad2d(edge_attr)

    mesh = plsc.VectorSubcoreMesh(core_axis_name="c", subcore_axis_name="s")

    sc_params = pltpu.CompilerParams(needs_layout_passes=False)

    agg_part = pl.kernel(
        _gcn_agg_body,
        out_type=jax.ShapeDtypeStruct((NC, npad, D), f32),
        mesh=mesh,
        compiler_params=sc_params,
        scratch_types=[
            pltpu.VMEM((8, 128), jnp.int32),
            pltpu.VMEM((8, 128), jnp.int32),
            pltpu.VMEM((8, 128), f32),
            pltpu.VMEM((128, D), f32),
            pltpu.VMEM((128, D), f32),
            pltpu.VMEM((128,), f32),
            pltpu.VMEM((npad,), f32),
            pltpu.VMEM((npad // NS,), f32),
            pltpu.VMEM_SHARED((npad,), f32),
            pltpu.VMEM_SHARED((npad, D), f32),
            pltpu.SemaphoreType.DMA,
            pltpu.SemaphoreType.DMA,
            pltpu.SemaphoreType.DMA,
        ],
    )(src_p, dst_p, ew_p, x)

    p1, p2 = pl.pallas_call(
        _dense_body,
        out_shape=(jax.ShapeDtypeStruct((npad, D), f32),
                   jax.ShapeDtypeStruct((npad, D), f32)),
    )(agg_part, conv_z_W, lin_z_W, lin_z_b.reshape(1, D),
      conv_z_b.reshape(1, D), conv_h_W, lin_h_W, lin_h_b.reshape(1, D),
      conv_h_b.reshape(1, D), dec1_W, dec1_b.reshape(1, D))

    s_p = pad2d(edge_weights_index[0])
    t_p = pad2d(edge_weights_index[1])
    w2 = dec2_W.reshape(D)
    b16 = jnp.pad(dec2_b, (0, L - dec2_b.shape[0]))

    pred2d = pl.kernel(
        _dec_body,
        out_type=jax.ShapeDtypeStruct((er, 128), f32),
        mesh=mesh,
        compiler_params=sc_params,
        scratch_types=[
            pltpu.VMEM((rpt, 128), jnp.int32),
            pltpu.VMEM((rpt, 128), jnp.int32),
            pltpu.VMEM((128, D), f32),
            pltpu.VMEM((128, D), f32),
            pltpu.VMEM((128, D), f32),
            pltpu.VMEM((128, D), f32),
            pltpu.VMEM((rpt, 128), f32),
            pltpu.VMEM((D,), f32),
            pltpu.VMEM((L,), f32),
            pltpu.SemaphoreType.DMA,
            pltpu.SemaphoreType.DMA,
            pltpu.SemaphoreType.DMA,
            pltpu.SemaphoreType.DMA,
        ],
    )(p1, p2, s_p, t_p, w2, b16)

    return pred2d.reshape(-1)[:e]
